# Initial kernel scaffold; baseline (speedup 1.0000x reference)
import jax
import jax.numpy as jnp
from jax.experimental import pallas as pl


def kernel(indices, weight):
    raise NotImplementedError("write your pallas kernel here")


# speedup vs baseline: 1.1101x; 1.1101x over previous
import functools

import jax
import jax.numpy as jnp
from jax import lax
from jax.experimental import pallas as pl
from jax.experimental.pallas import tpu as pltpu
from jax.experimental.pallas import tpu_sc as plsc

DICT_SIZE = 1000000
EMBD_SIZE = 32
N_SEQ = 16384
N_TOK = 50

_INFO = plsc.get_sparse_core_info()
NC, NS = _INFO.num_cores, _INFO.num_subcores
NW = NC * NS
B = N_SEQ * N_TOK
B_PER_W = B // NW
C = 128
NCH = B_PER_W // C
NBUF = 4
NGRP = NCH // NBUF

assert B % NW == 0 and B_PER_W % C == 0 and NCH % NBUF == 0

_mesh = plsc.VectorSubcoreMesh(core_axis_name="c", subcore_axis_name="s")


@functools.partial(
    pl.kernel,
    mesh=_mesh,
    out_type=jax.ShapeDtypeStruct((B, EMBD_SIZE), jnp.float32),
    scratch_types=[
        pltpu.VMEM((NCH, C), jnp.int32),
        pltpu.VMEM((C, EMBD_SIZE), jnp.float32),
        pltpu.VMEM((C, EMBD_SIZE), jnp.float32),
        pltpu.VMEM((C, EMBD_SIZE), jnp.float32),
        pltpu.VMEM((C, EMBD_SIZE), jnp.float32),
        pltpu.SemaphoreType.DMA,
        pltpu.SemaphoreType.DMA,
        pltpu.SemaphoreType.DMA,
        pltpu.SemaphoreType.DMA,
    ],
    compiler_params=pltpu.CompilerParams(use_tc_tiling_on_sc=False),
)
def _gather_kernel(idx_hbm, table_hbm, out_hbm,
                   idx_v, r0, r1, r2, r3, s0, s1, s2, s3):
    rows = (r0, r1, r2, r3)
    sems = (s0, s1, s2, s3)
    wid = lax.axis_index("s") * NC + lax.axis_index("c")
    base = wid * B_PER_W

    pltpu.sync_copy(idx_hbm.at[wid], idx_v)

    for b in range(NBUF):
        pltpu.async_copy(table_hbm.at[idx_v.at[b]], rows[b], sems[b])

    def group(g, carry):
        for b in range(NBUF):
            j = g * NBUF + b
            pltpu.make_async_copy(
                table_hbm.at[idx_v.at[j]], rows[b], sems[b]).wait()
            pltpu.sync_copy(rows[b], out_hbm.at[pl.ds(base + j * C, C)])
            pltpu.async_copy(
                table_hbm.at[idx_v.at[j + NBUF]], rows[b], sems[b])
        return carry

    lax.fori_loop(0, NGRP - 1, group, 0)

    for b in range(NBUF):
        j = (NGRP - 1) * NBUF + b
        pltpu.make_async_copy(
            table_hbm.at[idx_v.at[j]], rows[b], sems[b]).wait()
        pltpu.sync_copy(rows[b], out_hbm.at[pl.ds(base + j * C, C)])


def kernel(indices, weight):
    idx = indices.astype(jnp.int32).reshape(NW, NCH, C)
    out = _gather_kernel(idx, weight)
    return out.reshape(N_SEQ, N_TOK, EMBD_SIZE)


# speedup vs baseline: 1.7918x; 1.6141x over previous
import functools

import jax
import jax.numpy as jnp
from jax import lax
from jax.experimental import pallas as pl
from jax.experimental.pallas import tpu as pltpu
from jax.experimental.pallas import tpu_sc as plsc

DICT_SIZE = 1000000
EMBD_SIZE = 32
N_SEQ = 16384
N_TOK = 50

_INFO = plsc.get_sparse_core_info()
NC, NS = _INFO.num_cores, _INFO.num_subcores
NW = NC * NS
S_PER_W = N_SEQ // NW
NBUF = 8
NGRP = S_PER_W // NBUF

assert N_SEQ % NW == 0 and S_PER_W % NBUF == 0

_mesh = plsc.VectorSubcoreMesh(core_axis_name="c", subcore_axis_name="s")


@functools.partial(
    pl.kernel,
    mesh=_mesh,
    out_type=jax.ShapeDtypeStruct((N_SEQ, N_TOK, EMBD_SIZE), jnp.float32),
    scratch_types=[
        pltpu.VMEM((S_PER_W, N_TOK), jnp.int32),
        [pltpu.VMEM((N_TOK, EMBD_SIZE), jnp.float32)] * NBUF,
        [pltpu.SemaphoreType.DMA] * NBUF,
    ],
    compiler_params=pltpu.CompilerParams(use_tc_tiling_on_sc=False),
)
def _gather_kernel(idx_hbm, table_hbm, out_hbm, idx_v, rows, sems):
    wid = lax.axis_index("s") * NC + lax.axis_index("c")
    base = wid * S_PER_W

    pltpu.sync_copy(idx_hbm.at[pl.ds(base, S_PER_W)], idx_v)

    for b in range(NBUF):
        pltpu.async_copy(table_hbm.at[idx_v.at[b]], rows[b], sems[b])

    def group(g, carry):
        for b in range(NBUF):
            j = g * NBUF + b
            pltpu.make_async_copy(
                table_hbm.at[idx_v.at[j]], rows[b], sems[b]).wait()
            pltpu.sync_copy(rows[b], out_hbm.at[base + j])
            pltpu.async_copy(
                table_hbm.at[idx_v.at[j + NBUF]], rows[b], sems[b])
        return carry

    lax.fori_loop(0, NGRP - 1, group, 0)

    for b in range(NBUF):
        j = (NGRP - 1) * NBUF + b
        pltpu.make_async_copy(
            table_hbm.at[idx_v.at[j]], rows[b], sems[b]).wait()
        pltpu.sync_copy(rows[b], out_hbm.at[base + j])


def kernel(indices, weight):
    return _gather_kernel(indices.astype(jnp.int32), weight)


# speedup vs baseline: 1.7981x; 1.0035x over previous
import functools

import jax
import jax.numpy as jnp
from jax import lax
from jax.experimental import pallas as pl
from jax.experimental.pallas import tpu as pltpu
from jax.experimental.pallas import tpu_sc as plsc

DICT_SIZE = 1000000
EMBD_SIZE = 32
N_SEQ = 16384
N_TOK = 50

_INFO = plsc.get_sparse_core_info()
NC, NS = _INFO.num_cores, _INFO.num_subcores
NW = NC * NS
S_PER_W = N_SEQ // NW
NBUF = 16
NGRP = S_PER_W // NBUF

assert N_SEQ % NW == 0 and S_PER_W % NBUF == 0

_mesh = plsc.VectorSubcoreMesh(core_axis_name="c", subcore_axis_name="s")


@functools.partial(
    pl.kernel,
    mesh=_mesh,
    out_type=jax.ShapeDtypeStruct((N_SEQ, N_TOK, EMBD_SIZE), jnp.float32),
    scratch_types=[
        pltpu.VMEM((S_PER_W, N_TOK), jnp.int32),
        [pltpu.VMEM((N_TOK, EMBD_SIZE), jnp.float32)] * NBUF,
        [pltpu.SemaphoreType.DMA] * NBUF,
    ],
    compiler_params=pltpu.CompilerParams(use_tc_tiling_on_sc=False),
)
def _gather_kernel(idx_hbm, table_hbm, out_hbm, idx_v, rows, sems):
    wid = lax.axis_index("s") * NC + lax.axis_index("c")
    base = wid * S_PER_W

    pltpu.sync_copy(idx_hbm.at[pl.ds(base, S_PER_W)], idx_v)

    for b in range(NBUF):
        pltpu.async_copy(table_hbm.at[idx_v.at[b]], rows[b], sems[b])

    def group(g, carry):
        for b in range(NBUF):
            j = g * NBUF + b
            pltpu.make_async_copy(
                table_hbm.at[idx_v.at[j]], rows[b], sems[b]).wait()
            pltpu.sync_copy(rows[b], out_hbm.at[base + j])
            pltpu.async_copy(
                table_hbm.at[idx_v.at[j + NBUF]], rows[b], sems[b])
        return carry

    lax.fori_loop(0, NGRP - 1, group, 0)

    for b in range(NBUF):
        j = (NGRP - 1) * NBUF + b
        pltpu.make_async_copy(
            table_hbm.at[idx_v.at[j]], rows[b], sems[b]).wait()
        pltpu.sync_copy(rows[b], out_hbm.at[base + j])


def kernel(indices, weight):
    return _gather_kernel(indices.astype(jnp.int32), weight)
